# baseline (device time: 43719 ns/iter reference)
import os

import jax
import jax.numpy as jnp
from jax import lax
from jax.experimental import pallas as pl
from jax.experimental.pallas import tpu as pltpu

N_DEV = 4
B, S, D = 2, 512, 768
BS = B * S
H_LOC = 4
DH = 96
SCALE = 0.10206207261596577
LOG2E = 1.4426950408889634
EPS = 1e-5
BF = jnp.bfloat16
F32 = jnp.float32
F8 = jnp.float8_e4m3fn
_COMM = os.environ.get("KCOMM", "1") == "1"


def kernel(x, Wq, Wk, Wv, Wo, t_emb, W_mod, W_ff1, W_ff2):
    def body(x_ref, wq_ref, wk_ref, wv_ref, wo_ref, temb_ref, wmod_ref,
             wff1_ref, wff2_ref, out_ref,
             acc_ref, g1_ref, g2_ref, rbig, rmed, q_ref, ssem, rsem):
        my = lax.axis_index("i")
        xc = (my // 2).astype(jnp.int32)
        yc = jnp.where((my == 1) | (my == 2), 1, 0).astype(jnp.int32)
        pA = my + 1 - 2 * lax.rem(my, 2)
        pB = 3 - my

        if _COMM:
            barrier = pltpu.get_barrier_semaphore()
            for nbr in (pA, pB):
                pl.semaphore_signal(
                    barrier, inc=1,
                    device_id=(nbr,), device_id_type=pl.DeviceIdType.MESH,
                )
            pl.semaphore_wait(barrier, 2)

        P = [[pA, pB], [pB, pA]]
        C = [[yc, xc], [xc, yc]]
        c0 = [C[h][0] for h in range(2)]
        keep_off = [c0[h] * 256 for h in range(2)]
        send_off = [(1 - c0[h]) * 256 for h in range(2)]

        deferred = []

        def start_stage(ref, offs, n, level, col, base, dsts=None):
            rds = []
            if not _COMM:
                return rds
            for h in range(2):
                src = ref.at[pl.ds(h * 512 + offs[h], n)]
                dst = dsts.at[h] if dsts is not None else src
                r = pltpu.make_async_remote_copy(
                    src_ref=src, dst_ref=dst,
                    send_sem=ssem.at[base + h, col],
                    recv_sem=rsem.at[base + h, col],
                    device_id=(P[h][level],),
                    device_id_type=pl.DeviceIdType.MESH,
                )
                r.start()
                rds.append(r)
                deferred.append(r)
            return rds

        def wait_recvs(rds):
            for r in rds:
                r.wait_recv()

        def ln(h):
            m = jnp.mean(h, axis=-1, keepdims=True)
            v = jnp.mean((h - m) * (h - m), axis=-1, keepdims=True)
            return (h - m) * lax.rsqrt(v + EPS)

        mod = jnp.dot(temb_ref[...], wmod_ref[...],
                      preferred_element_type=F32)
        sa, sha, ga = mod[:, 0:D], mod[:, D:2 * D], mod[:, 2 * D:3 * D]
        sm, shm, gm = mod[:, 3 * D:4 * D], mod[:, 4 * D:5 * D], mod[:, 5 * D:]

        x0 = x_ref[...]
        xm = ln(x0) * (1.0 + sa[:, None, :]) + sha[:, None, :]
        xm2d = xm.reshape(BS, D).astype(BF)

        wq_b = wq_ref[...].astype(BF)
        wk_b = wk_ref[...].astype(BF)
        wv_b = wv_ref[...].astype(BF)
        wo_b = wo_ref[...].astype(BF)
        q_ref[...] = (
            jnp.dot(xm2d, wq_b, preferred_element_type=F32) * (SCALE * LOG2E)
        ).astype(BF)
        k = jnp.dot(xm2d, wk_b, preferred_element_type=F32).astype(BF)
        v = jnp.dot(xm2d, wv_b, preferred_element_type=F32).astype(BF)

        def attn_wo(h, s0, n):
            b0 = h * 512
            qrows = q_ref[pl.ds(b0 + s0, n)]
            outs = []
            for hd in range(H_LOC):
                qh = qrows[:, hd * DH:(hd + 1) * DH]
                kh = k[b0:b0 + S, hd * DH:(hd + 1) * DH]
                vh = v[b0:b0 + S, hd * DH:(hd + 1) * DH]
                s_ = lax.dot_general(
                    qh, kh, (((1,), (1,)), ((), ())),
                    preferred_element_type=F32,
                )
                p = jnp.exp2(s_)
                l = jnp.sum(p, axis=-1, keepdims=True)
                o = jnp.dot(p.astype(BF), vh, preferred_element_type=F32)
                outs.append(o / l)
            a = jnp.concatenate(outs, axis=1)
            return jnp.dot(a.astype(BF), wo_b, preferred_element_type=F32)

        wff1_b = wff1_ref[...].astype(BF)
        wff2_b = wff2_ref[...].astype(BF)

        def ffn_rows(h, s0, n):
            off = h * 512 + s0
            xr = (x_ref[h, pl.ds(s0, n), :]
                  + g1_ref[pl.ds(off, n)].astype(F32))
            xr = ln(xr) * (1.0 + sm[h:h + 1, :]) + shm[h:h + 1, :]
            hp = jnp.dot(xr.astype(BF), wff1_b, preferred_element_type=F32)
            ha = hp * (1.0 / (1.0 + jnp.exp(-hp)))
            p2 = jnp.dot(ha.astype(BF), wff2_b, preferred_element_type=F32)
            acc_ref[pl.ds(off, n)] = p2.astype(F8)

        def out_rows(h, s0, n):
            off = h * 512 + s0
            out_ref[h, pl.ds(s0, n), :] = (
                x_ref[h, pl.ds(s0, n), :]
                + g1_ref[pl.ds(off, n)].astype(F32)
                + g2_ref[pl.ds(off, n)].astype(F32))

        for h in range(2):
            acc_ref[pl.ds(h * 512 + send_off[h], 256)] = (
                attn_wo(h, send_off[h], 256).astype(F8))
        rds0 = start_stage(acc_ref, send_off, 256, 0, 0, 0, dsts=rbig)
        for h in range(2):
            acc_ref[pl.ds(h * 512 + keep_off[h], 256)] = (
                attn_wo(h, keep_off[h], 256).astype(F8))
        wait_recvs(rds0)
        for h in range(2):
            acc_ref[pl.ds(h * 512 + keep_off[h], 256)] = (
                acc_ref[pl.ds(h * 512 + keep_off[h], 256)].astype(F32)
                + rbig[h].astype(F32)).astype(F8)

        rds1 = start_stage(acc_ref, keep_off, 256, 1, 1, 0, dsts=rmed)
        wait_recvs(rds1)
        for h in range(2):
            off = h * 512 + keep_off[h]
            piece = (acc_ref[pl.ds(off, 256)].astype(F32)
                     + rmed[h].astype(F32))
            g1_ref[pl.ds(off, 256)] = (ga[h:h + 1, :] * piece).astype(F8)

        rds2 = start_stage(g1_ref, keep_off, 256, 0, 2, 0)
        for h in range(2):
            ffn_rows(h, keep_off[h], 256)
        wait_recvs(rds2)

        rds0b = start_stage(acc_ref, keep_off, 256, 0, 0, 2, dsts=rbig)
        for h in range(2):
            ffn_rows(h, send_off[h], 256)
        wait_recvs(rds0b)
        for h in range(2):
            acc_ref[pl.ds(h * 512 + send_off[h], 256)] = (
                acc_ref[pl.ds(h * 512 + send_off[h], 256)].astype(F32)
                + rbig[h].astype(F32)).astype(F8)

        rds1b = start_stage(acc_ref, send_off, 256, 1, 1, 2, dsts=rmed)
        wait_recvs(rds1b)
        for h in range(2):
            off = h * 512 + send_off[h]
            piece = (acc_ref[pl.ds(off, 256)].astype(F32)
                     + rmed[h].astype(F32))
            g2_ref[pl.ds(off, 256)] = (gm[h:h + 1, :] * piece).astype(F8)

        rds2b = start_stage(g2_ref, send_off, 256, 0, 2, 2)
        for h in range(2):
            out_rows(h, send_off[h], 256)
        wait_recvs(rds2b)
        for h in range(2):
            out_rows(h, keep_off[h], 256)

        for r in deferred:
            r.wait_send()

    return pl.pallas_call(
        body,
        out_shape=jax.ShapeDtypeStruct((B, S, D), F32),
        in_specs=[pl.BlockSpec(memory_space=pltpu.VMEM)] * 9,
        out_specs=pl.BlockSpec(memory_space=pltpu.VMEM),
        scratch_shapes=[
            pltpu.VMEM((BS, D), F8),
            pltpu.VMEM((BS, D), F8),
            pltpu.VMEM((BS, D), F8),
            pltpu.VMEM((2, 256, D), F8),
            pltpu.VMEM((2, 256, D), F8),
            pltpu.VMEM((BS, H_LOC * DH), BF),
            pltpu.SemaphoreType.DMA((4, 3)),
            pltpu.SemaphoreType.DMA((4, 3)),
        ],
        compiler_params=pltpu.CompilerParams(
            collective_id=0 if _COMM else None,
            vmem_limit_bytes=128 * 1024 * 1024,
        ),
    )(x, Wq, Wk, Wv, Wo, t_emb, W_mod, W_ff1, W_ff2)
